# baseline (speedup 1.0000x reference)
import functools

import jax
import jax.numpy as jnp
from jax import lax
from jax.experimental import pallas as pl
from jax.experimental.pallas import tpu as pltpu
from jax.experimental.pallas import tpu_sc as plsc

_NC = 2
_NS = 16
_NW = _NC * _NS
_CHUNK = 8192


def _sc_gather_t(tab2, xt):
    fd, v = tab2.shape
    f, b = xt.shape
    ha = (v // 2 // 128) * 128
    hb = v - ha
    n_jobs = fd // _NW
    oc = 4096
    nch = b // oc
    mesh = plsc.VectorSubcoreMesh(core_axis_name="c", subcore_axis_name="s")

    @functools.partial(
        pl.kernel,
        out_type=jax.ShapeDtypeStruct((fd, b), jnp.float32),
        mesh=mesh,
        compiler_params=pltpu.CompilerParams(needs_layout_passes=False),
        scratch_types=[
            pltpu.VMEM((ha,), jnp.float32),
            pltpu.VMEM((hb,), jnp.float32),
            pltpu.VMEM((oc,), jnp.int32),
            pltpu.VMEM((oc,), jnp.int32),
            pltpu.VMEM((b,), jnp.float32),
            pltpu.VMEM_SHARED((b,), jnp.int32),
            pltpu.VMEM_SHARED((b,), jnp.int32),
            pltpu.SemaphoreType.DMA,
            pltpu.SemaphoreType.DMA,
            pltpu.SemaphoreType.DMA,
            pltpu.SemaphoreType.DMA,
            pltpu.SemaphoreType.DMA,
        ],
    )
    def gather_k(tab_hbm, xt_hbm, out_hbm, row_a, row_b, idx_e, idx_o, outv,
                 spi_e, spi_o, sem_a, sem_b, sem_i, sem_o, sem_x):
        cid = lax.axis_index("c")
        sid = lax.axis_index("s")
        wid = sid * _NC + cid
        iota16 = lax.iota(jnp.int32, 16)

        pltpu.async_copy(tab_hbm.at[wid, pl.ds(0, ha)], row_a, sem_a)
        pltpu.async_copy(tab_hbm.at[wid, pl.ds(ha, hb)], row_b, sem_b)

        @pl.when(sid == 0)
        def _prime_idx():
            pltpu.async_copy(xt_hbm.at[0], spi_e, sem_x).wait()
        plsc.subcore_barrier()
        pltpu.async_copy(spi_e.at[pl.ds(0, oc)], idx_e, sem_i)

        def gather_chunk(spi, hi, c):
            gc = (4 if hi else 0) + c
            base = c * oc
            ib = idx_e if gc % 2 == 0 else idx_o
            pltpu.make_async_copy(spi.at[pl.ds(0, oc)], ib, sem_i).wait()
            if gc < 2 * nch - 1:
                nxt = idx_o if gc % 2 == 0 else idx_e
                ncol = ((gc + 1) % nch) * oc
                pltpu.async_copy(spi.at[pl.ds(ncol, oc)], nxt, sem_i)

            @plsc.parallel_loop(0, oc, step=16, unroll=16)
            def _g(i):
                iv = ib[pl.ds(i, 16)]
                if hi:
                    vc = jnp.maximum(iv - ha, 0)
                    val = plsc.load_gather(row_b, [vc])
                    pos = iota16 + (base + i)
                    plsc.store_scatter(outv, [pos], val, mask=iv >= ha)
                else:
                    sl = pl.ds(base + i, 16)
                    vc = jnp.minimum(iv, ha - 1)
                    outv[sl] = plsc.load_gather(row_a, [vc])

        def job(j, spi, spi_next, carry):
            r = j * _NW + wid
            r_next = r + _NW

            @pl.when(jnp.logical_and(sid == 0, j < n_jobs - 1))
            def _pf_idx():
                pltpu.async_copy(xt_hbm.at[j + 1], spi_next, sem_x)

            @pl.when(j > 0)
            def _drain():
                def w(c, cc):
                    pltpu.make_async_copy(
                        outv.at[pl.ds(0, 2 * oc)],
                        out_hbm.at[r, pl.ds(0, 2 * oc)], sem_o).wait()
                    return cc
                lax.fori_loop(0, nch // 2, w, 0)

            pltpu.make_async_copy(tab_hbm.at[r, pl.ds(0, ha)], row_a,
                                  sem_a).wait()
            for c in range(nch):
                gather_chunk(spi, False, c)

            @pl.when(j < n_jobs - 1)
            def _pf_a():
                pltpu.async_copy(tab_hbm.at[r_next, pl.ds(0, ha)], row_a,
                                 sem_a)

            pltpu.make_async_copy(tab_hbm.at[r, pl.ds(ha, hb)], row_b,
                                  sem_b).wait()
            for c in range(nch):
                gather_chunk(spi, True, c)
                if c % 2 == 1:
                    pltpu.async_copy(
                        outv.at[pl.ds((c - 1) * oc, 2 * oc)],
                        out_hbm.at[r, pl.ds((c - 1) * oc, 2 * oc)], sem_o)

            @pl.when(j < n_jobs - 1)
            def _pf_b():
                pltpu.async_copy(tab_hbm.at[r_next, pl.ds(ha, hb)], row_b,
                                 sem_b)

            @pl.when(jnp.logical_and(sid == 0, j < n_jobs - 1))
            def _wait_idx():
                pltpu.make_async_copy(xt_hbm.at[0], spi_next, sem_x).wait()
            plsc.subcore_barrier()
            @pl.when(j < n_jobs - 1)
            def _pf_c0():
                pltpu.async_copy(spi_next.at[pl.ds(0, oc)], idx_e, sem_i)
            return carry

        def job_pair(p, carry):
            job(2 * p, spi_e, spi_o, carry)
            job(2 * p + 1, spi_o, spi_e, carry)
            return carry

        lax.fori_loop(0, n_jobs // 2, job_pair, 0)
        for _c in range(nch // 2):
            pltpu.make_async_copy(outv.at[pl.ds(0, 2 * oc)],
                                  out_hbm.at[0, pl.ds(0, 2 * oc)], sem_o).wait()

    return gather_k(tab2, xt)


def _mlp_body(embt_ref, s_ref, w1_ref, b1_ref, w2_ref, b2_ref, w3_ref, b3_ref,
              out_ref):
    dn = (((0,), (0,)), ((), ()))
    ft = embt_ref[...]
    sp = lax.dot_general(s_ref[...], ft, dn,
                         preferred_element_type=jnp.float32)
    left = jnp.sum(sp * sp, axis=0, keepdims=True)
    right = jnp.sum(ft * ft, axis=0, keepdims=True)
    fm = 0.5 * (left - right)
    h = lax.dot_general(w1_ref[...], ft, dn,
                        preferred_element_type=jnp.float32)
    h = jnp.maximum(h + b1_ref[...], 0.0)
    h = lax.dot_general(w2_ref[...], h, dn,
                        preferred_element_type=jnp.float32)
    h = jnp.maximum(h + b2_ref[...], 0.0)
    h = lax.dot_general(w3_ref[...], h, dn,
                        preferred_element_type=jnp.float32)
    h = jnp.maximum(h + b3_ref[...], 0.0)
    out_ref[...] = (fm + h)[:8]


def _tc_mlp_t(embt, s, w1, b1c, w2p, b2c, w3p, b3c, bb):
    din, b = embt.shape
    grid = (b // bb,)
    return pl.pallas_call(
        _mlp_body,
        grid=grid,
        in_specs=[
            pl.BlockSpec((din, bb), lambda i: (0, i)),
            pl.BlockSpec(s.shape, lambda i: (0, 0)),
            pl.BlockSpec(w1.shape, lambda i: (0, 0)),
            pl.BlockSpec(b1c.shape, lambda i: (0, 0)),
            pl.BlockSpec(w2p.shape, lambda i: (0, 0)),
            pl.BlockSpec(b2c.shape, lambda i: (0, 0)),
            pl.BlockSpec(w3p.shape, lambda i: (0, 0)),
            pl.BlockSpec(b3c.shape, lambda i: (0, 0)),
        ],
        out_specs=pl.BlockSpec((8, bb), lambda i: (0, i)),
        out_shape=jax.ShapeDtypeStruct((8, b), jnp.float32),
    )(embt, s, w1, b1c, w2p, b2c, w3p, b3c)


def kernel(x, tables, W1, b1, W2, b2, W3, b3):
    B, F = x.shape
    V, D = tables.shape[1], tables.shape[2]

    tab2 = tables.transpose(0, 2, 1).reshape(F * D, V)
    xt = x.T.astype(jnp.int32)

    embt = _sc_gather_t(tab2, xt)

    s = jnp.tile(jnp.eye(D, dtype=jnp.float32), (F, 1))
    b1c = b1.reshape(128, 1)
    w2p = jnp.pad(W2, ((0, 0), (0, 128 - W2.shape[1])))
    b2c = jnp.pad(b2, (0, 128 - b2.shape[0])).reshape(128, 1)
    w3p = jnp.pad(W3, ((0, 128 - W3.shape[0]), (0, 128 - W3.shape[1])))
    b3c = jnp.pad(b3, (0, 128 - b3.shape[0])).reshape(128, 1)

    out_t = _tc_mlp_t(embt, s, W1, b1c, w2p, b2c, w3p, b3c, bb=2048)
    return out_t[:2, :].T


# speedup vs baseline: 1.0321x; 1.0321x over previous
import functools

import jax
import jax.numpy as jnp
from jax import lax
from jax.experimental import pallas as pl
from jax.experimental.pallas import tpu as pltpu
from jax.experimental.pallas import tpu_sc as plsc

_NC = 2
_NS = 16
_NW = _NC * _NS
_CHUNK = 8192


def _sc_gather_t(tab2, xt):
    fd, v = tab2.shape
    f, b = xt.shape
    ha = (v // 2 // 128) * 128
    hb = v - ha
    n_jobs = fd // _NW
    oc = 4096
    nch = b // oc
    mesh = plsc.VectorSubcoreMesh(core_axis_name="c", subcore_axis_name="s")

    @functools.partial(
        pl.kernel,
        out_type=jax.ShapeDtypeStruct((fd, b), jnp.float32),
        mesh=mesh,
        compiler_params=pltpu.CompilerParams(needs_layout_passes=False),
        scratch_types=[
            pltpu.VMEM((ha,), jnp.float32),
            pltpu.VMEM((hb,), jnp.float32),
            pltpu.VMEM((oc,), jnp.int32),
            pltpu.VMEM((oc,), jnp.int32),
            pltpu.VMEM((b,), jnp.float32),
            pltpu.VMEM_SHARED((b,), jnp.int32),
            pltpu.VMEM_SHARED((b,), jnp.int32),
            pltpu.SemaphoreType.DMA,
            pltpu.SemaphoreType.DMA,
            pltpu.SemaphoreType.DMA,
            pltpu.SemaphoreType.DMA,
            pltpu.SemaphoreType.DMA,
        ],
    )
    def gather_k(tab_hbm, xt_hbm, out_hbm, row_a, row_b, idx_e, idx_o, outv,
                 spi_e, spi_o, sem_a, sem_b, sem_i, sem_o, sem_x):
        cid = lax.axis_index("c")
        sid = lax.axis_index("s")
        wid = sid * _NC + cid
        iota16 = lax.iota(jnp.int32, 16)

        pltpu.async_copy(tab_hbm.at[wid, pl.ds(0, ha)], row_a, sem_a)
        pltpu.async_copy(tab_hbm.at[wid, pl.ds(ha, hb)], row_b, sem_b)

        @pl.when(sid == 0)
        def _prime_idx():
            pltpu.async_copy(xt_hbm.at[0], spi_e, sem_x).wait()
        plsc.subcore_barrier()
        pltpu.async_copy(spi_e.at[pl.ds(0, oc)], idx_e, sem_i)

        def gather_chunk(spi, hi, c):
            gc = (4 if hi else 0) + c
            base = c * oc
            ib = idx_e if gc % 2 == 0 else idx_o
            pltpu.make_async_copy(spi.at[pl.ds(0, oc)], ib, sem_i).wait()
            if gc < 2 * nch - 1:
                nxt = idx_o if gc % 2 == 0 else idx_e
                ncol = ((gc + 1) % nch) * oc
                pltpu.async_copy(spi.at[pl.ds(ncol, oc)], nxt, sem_i)

            @plsc.parallel_loop(0, oc, step=16, unroll=8)
            def _g(i):
                iv = ib[pl.ds(i, 16)]
                if hi:
                    vc = jnp.maximum(iv - ha, 0)
                    val = plsc.load_gather(row_b, [vc])
                    pos = iota16 + (base + i)
                    plsc.store_scatter(outv, [pos], val, mask=iv >= ha)
                else:
                    sl = pl.ds(base + i, 16)
                    vc = jnp.minimum(iv, ha - 1)
                    outv[sl] = plsc.load_gather(row_a, [vc])

        def job(j, spi, spi_next, carry):
            r = j * _NW + wid
            r_next = r + _NW

            @pl.when(jnp.logical_and(sid == 0, j < n_jobs - 1))
            def _pf_idx():
                pltpu.async_copy(xt_hbm.at[j + 1], spi_next, sem_x)

            @pl.when(j > 0)
            def _drain():
                def w(c, cc):
                    pltpu.make_async_copy(
                        outv.at[pl.ds(0, oc)],
                        out_hbm.at[r, pl.ds(0, oc)], sem_o).wait()
                    return cc
                lax.fori_loop(0, nch, w, 0)

            pltpu.make_async_copy(tab_hbm.at[r, pl.ds(0, ha)], row_a,
                                  sem_a).wait()
            for c in range(nch):
                gather_chunk(spi, False, c)

            @pl.when(j < n_jobs - 1)
            def _pf_a():
                pltpu.async_copy(tab_hbm.at[r_next, pl.ds(0, ha)], row_a,
                                 sem_a)

            pltpu.make_async_copy(tab_hbm.at[r, pl.ds(ha, hb)], row_b,
                                  sem_b).wait()
            for c in range(nch):
                gather_chunk(spi, True, c)
                pltpu.async_copy(outv.at[pl.ds(c * oc, oc)],
                                 out_hbm.at[r, pl.ds(c * oc, oc)], sem_o)

            @pl.when(j < n_jobs - 1)
            def _pf_b():
                pltpu.async_copy(tab_hbm.at[r_next, pl.ds(ha, hb)], row_b,
                                 sem_b)

            @pl.when(jnp.logical_and(sid == 0, j < n_jobs - 1))
            def _wait_idx():
                pltpu.make_async_copy(xt_hbm.at[0], spi_next, sem_x).wait()
            plsc.subcore_barrier()
            @pl.when(j < n_jobs - 1)
            def _pf_c0():
                pltpu.async_copy(spi_next.at[pl.ds(0, oc)], idx_e, sem_i)
            return carry

        def job_pair(p, carry):
            job(2 * p, spi_e, spi_o, carry)
            job(2 * p + 1, spi_o, spi_e, carry)
            return carry

        lax.fori_loop(0, n_jobs // 2, job_pair, 0)
        for _c in range(nch):
            pltpu.make_async_copy(outv.at[pl.ds(0, oc)],
                                  out_hbm.at[0, pl.ds(0, oc)], sem_o).wait()

    return gather_k(tab2, xt)


def _mlp_body(embt_ref, s_ref, w1_ref, b1_ref, w2_ref, b2_ref, w3_ref, b3_ref,
              out_ref):
    dn = (((0,), (0,)), ((), ()))
    ft = embt_ref[...]
    sp = lax.dot_general(s_ref[...], ft, dn,
                         preferred_element_type=jnp.float32)
    left = jnp.sum(sp * sp, axis=0, keepdims=True)
    right = jnp.sum(ft * ft, axis=0, keepdims=True)
    fm = 0.5 * (left - right)
    h = lax.dot_general(w1_ref[...], ft, dn,
                        preferred_element_type=jnp.float32)
    h = jnp.maximum(h + b1_ref[...], 0.0)
    h = lax.dot_general(w2_ref[...], h, dn,
                        preferred_element_type=jnp.float32)
    h = jnp.maximum(h + b2_ref[...], 0.0)
    h = lax.dot_general(w3_ref[...], h, dn,
                        preferred_element_type=jnp.float32)
    h = jnp.maximum(h + b3_ref[...], 0.0)
    out_ref[...] = (fm + h)[:8]


def _tc_mlp_t(embt, s, w1, b1c, w2p, b2c, w3p, b3c, bb):
    din, b = embt.shape
    grid = (b // bb,)
    return pl.pallas_call(
        _mlp_body,
        grid=grid,
        in_specs=[
            pl.BlockSpec((din, bb), lambda i: (0, i)),
            pl.BlockSpec(s.shape, lambda i: (0, 0)),
            pl.BlockSpec(w1.shape, lambda i: (0, 0)),
            pl.BlockSpec(b1c.shape, lambda i: (0, 0)),
            pl.BlockSpec(w2p.shape, lambda i: (0, 0)),
            pl.BlockSpec(b2c.shape, lambda i: (0, 0)),
            pl.BlockSpec(w3p.shape, lambda i: (0, 0)),
            pl.BlockSpec(b3c.shape, lambda i: (0, 0)),
        ],
        out_specs=pl.BlockSpec((8, bb), lambda i: (0, i)),
        out_shape=jax.ShapeDtypeStruct((8, b), jnp.float32),
    )(embt, s, w1, b1c, w2p, b2c, w3p, b3c)


def kernel(x, tables, W1, b1, W2, b2, W3, b3):
    B, F = x.shape
    V, D = tables.shape[1], tables.shape[2]

    tab2 = tables.transpose(0, 2, 1).reshape(F * D, V)
    xt = x.T.astype(jnp.int32)

    embt = _sc_gather_t(tab2, xt)

    s = jnp.tile(jnp.eye(D, dtype=jnp.float32), (F, 1))
    b1c = b1.reshape(128, 1)
    w2p = jnp.pad(W2, ((0, 0), (0, 128 - W2.shape[1])))
    b2c = jnp.pad(b2, (0, 128 - b2.shape[0])).reshape(128, 1)
    w3p = jnp.pad(W3, ((0, 128 - W3.shape[0]), (0, 128 - W3.shape[1])))
    b3c = jnp.pad(b3, (0, 128 - b3.shape[0])).reshape(128, 1)

    out_t = _tc_mlp_t(embt, s, W1, b1c, w2p, b2c, w3p, b3c, bb=2048)
    return out_t[:2, :].T
